# Initial kernel scaffold; baseline (speedup 1.0000x reference)
import jax
import jax.numpy as jnp
from jax.experimental import pallas as pl


def kernel(node_feats, edge_index, node_graph_ids, target, dist, W1, al1, ar1, b1, W2, al2, ar2, b2, atomW, atomb, protW, protb, distW, distb, d1W, d1b, d2W, d2b, outW, outb):
    raise NotImplementedError("write your pallas kernel here")


# speedup vs baseline: 1.1511x; 1.1511x over previous
import jax
import jax.numpy as jnp
from jax.experimental import pallas as pl
from jax.experimental.pallas import tpu as pltpu

_B = 512
_NPG = 32
_N = _B * _NPG
_H1, _F1 = 10, 74
_F2 = 128


def _head_body(h2r_ref, target_ref, dist_ref, atomW_ref, atomb_ref,
               protW_ref, protb_ref, distW_ref, distb_ref,
               d1W_ref, d1b_ref, d2W_ref, d2b_ref, outW_ref, outb_ref,
               out_ref):
    h2r = h2r_ref[...]
    atomW = atomW_ref[...]
    logits = jax.lax.dot_general(
        h2r.reshape(_B * _NPG, _F2), atomW,
        (((1,), (0,)), ((), ())), preferred_element_type=jnp.float32)
    w = jax.nn.sigmoid(logits + atomb_ref[0, 0]).reshape(_B, _NPG, 1)
    hsum = jnp.sum(h2r * w, axis=1)
    hmax = jnp.max(h2r, axis=1)

    x_prot = jax.lax.dot_general(
        target_ref[...], protW_ref[...],
        (((1,), (0,)), ((), ())), preferred_element_type=jnp.float32)
    x_prot = x_prot + protb_ref[...]
    x_dist = jax.lax.dot_general(
        dist_ref[...], distW_ref[...],
        (((1,), (0,)), ((), ())), preferred_element_type=jnp.float32)
    x_dist = x_dist + distb_ref[...]

    x = jnp.concatenate([x_prot, hsum, hmax, x_dist], axis=1)
    x = jax.lax.dot_general(x, d1W_ref[...],
                            (((1,), (0,)), ((), ())),
                            preferred_element_type=jnp.float32)
    x = jnp.maximum(x + d1b_ref[...], 0.0)
    x = jax.lax.dot_general(x, d2W_ref[...],
                            (((1,), (0,)), ((), ())),
                            preferred_element_type=jnp.float32)
    x = jnp.maximum(x + d2b_ref[...], 0.0)
    out = jax.lax.dot_general(x, outW_ref[...],
                              (((1,), (0,)), ((), ())),
                              preferred_element_type=jnp.float32)
    out_ref[...] = out + outb_ref[0, 0]


def _head(h2, target, dist, atomW, atomb, protW, protb, distW, distb,
          d1W, d1b, d2W, d2b, outW, outb):
    h2r = h2.reshape(_B, _NPG, _F2)
    out = pl.pallas_call(
        _head_body,
        out_shape=jax.ShapeDtypeStruct((_B, 1), jnp.float32),
    )(h2r, target, dist, atomW, atomb.reshape(1, 1),
      protW, protb.reshape(1, -1), distW, distb.reshape(1, -1),
      d1W, d1b.reshape(1, -1), d2W, d2b.reshape(1, -1),
      outW, outb.reshape(1, 1))
    return out[:, 0]


def _gat_layer(x, src, dst, W, al, ar, b, H, F, agg):
    n = x.shape[0]
    feat = (x @ W).reshape(n, H, F)
    el = jnp.sum(feat * al[None, :, :], axis=-1)
    er = jnp.sum(feat * ar[None, :, :], axis=-1)
    c = jax.nn.leaky_relu(jnp.max(el) + jnp.max(er), 0.2)
    e = jax.nn.leaky_relu(el[src] + er[dst], 0.2)
    ee = jnp.exp(e - c)
    denom = jax.ops.segment_sum(ee, dst, num_segments=n)
    num = jax.ops.segment_sum(feat[src] * ee[:, :, None], dst, num_segments=n)
    out = num / (denom[:, :, None] + 1e-9)
    out = out + b.reshape(1, H, F)
    if agg == 'flatten':
        out = out.reshape(n, H * F)
    else:
        out = out.mean(axis=1)
    return jax.nn.elu(out)


def kernel(node_feats, edge_index, node_graph_ids, target, dist, W1, al1,
           ar1, b1, W2, al2, ar2, b2, atomW, atomb, protW, protb, distW,
           distb, d1W, d1b, d2W, d2b, outW, outb):
    src, dst = edge_index[0], edge_index[1]
    h = _gat_layer(node_feats, src, dst, W1, al1, ar1, b1, _H1, _F1,
                   'flatten')
    h = _gat_layer(h, src, dst, W2, al2, ar2, b2, 1, _F2, 'mean')
    return _head(h, target, dist, atomW, atomb, protW, protb, distW,
                 distb, d1W, d1b, d2W, d2b, outW, outb)


# speedup vs baseline: 1.2853x; 1.1166x over previous
import functools

import jax
import jax.numpy as jnp
from jax import lax
from jax.experimental import pallas as pl
from jax.experimental.pallas import tpu as pltpu
from jax.experimental.pallas import tpu_sc as plsc

_B = 512
_NPG = 32
_N = _B * _NPG
_E = _N * 3
_H1, _F1 = 10, 74
_F2 = 128
_CH = 2048
_NW = 32
_NODES_PER_TILE = _N // _NW


def _lrelu(x):
    return jnp.maximum(x, 0.2 * x)


def _proj2_body(h_ref, W2_ref, al2_ref, ar2_ref, feat2_ref, el2_ref,
                er2_ref, pmax_ref):
    feat2 = jax.lax.dot_general(
        h_ref[...], W2_ref[...], (((1,), (0,)), ((), ())),
        preferred_element_type=jnp.float32)
    feat2_ref[...] = feat2
    el2 = jax.lax.dot_general(
        feat2, al2_ref[...], (((1,), (1,)), ((), ())),
        preferred_element_type=jnp.float32)
    er2 = jax.lax.dot_general(
        feat2, ar2_ref[...], (((1,), (1,)), ((), ())),
        preferred_element_type=jnp.float32)
    el2_ref[...] = el2
    er2_ref[...] = er2
    lane = lax.broadcasted_iota(jnp.int32, (1, 1, 16), 2)
    mel = jnp.max(el2)
    mer = jnp.max(er2)
    pmax_ref[...] = jnp.where(lane == 0, mel,
                              jnp.where(lane == 1, mer, -1e30))


def _proj2(h, W2, al2, ar2):
    nblk = 32
    blk = _N // nblk
    return pl.pallas_call(
        _proj2_body,
        grid=(nblk,),
        in_specs=[
            pl.BlockSpec((blk, _H1 * _F1), lambda g: (g, 0)),
            pl.BlockSpec((_H1 * _F1, _F2), lambda g: (0, 0)),
            pl.BlockSpec((1, _F2), lambda g: (0, 0)),
            pl.BlockSpec((1, _F2), lambda g: (0, 0)),
        ],
        out_specs=[
            pl.BlockSpec((blk, _F2), lambda g: (g, 0)),
            pl.BlockSpec((blk, 1), lambda g: (g, 0)),
            pl.BlockSpec((blk, 1), lambda g: (g, 0)),
            pl.BlockSpec((1, 1, 16), lambda g: (g, 0, 0)),
        ],
        out_shape=[
            jax.ShapeDtypeStruct((_N, _F2), jnp.float32),
            jax.ShapeDtypeStruct((_N, 1), jnp.float32),
            jax.ShapeDtypeStruct((_N, 1), jnp.float32),
            jax.ShapeDtypeStruct((nblk, 1, 16), jnp.float32),
        ],
    )(h, W2, al2, ar2)


def _sc_l2_body(feat2, el2h, er2h, pmaxh, srch, dsth,
                z2, den2,
                acc, den, el2v, er2v, dstc, srcc, msrc, mloc, mee,
                xbuf, pmaxb, sem1):
    cc = lax.axis_index("c")
    ss = lax.axis_index("s")
    w = ss * 2 + cc
    base = w * _NODES_PER_TILE
    lane = lax.broadcasted_iota(jnp.int32, (16,), 0)
    z16 = jnp.zeros((16,), jnp.float32)

    pltpu.sync_copy(el2h, el2v)
    pltpu.sync_copy(er2h, er2v)
    pltpu.sync_copy(pmaxh, pmaxb)

    def _pm(i, mv):
        return jnp.maximum(mv, pmaxb[pl.ds(i * 16, 16)])
    mv = lax.fori_loop(1, 32, _pm, pmaxb[pl.ds(0, 16)])
    cv = jnp.full((16,), _lrelu(mv[0] + mv[1]), jnp.float32)

    def _zero(i, _):
        for k in range(8):
            acc[pl.ds(i * 128 + k * 16, 16)] = z16
        den[pl.ds(i * 16, 16)] = z16
        return 0
    lax.fori_loop(0, _NODES_PER_TILE + 1, _zero, 0)

    def chunk_body(ch, _):
        pltpu.sync_copy(dsth.at[pl.ds(ch * _CH, _CH)], dstc)
        pltpu.sync_copy(srch.at[pl.ds(ch * _CH, _CH)], srcc)

        def scan_body(v, mcount):
            dstv = dstc[pl.ds(v * 16, 16)]
            srcv = srcc[pl.ds(v * 16, 16)]
            lv = dstv - base
            m = (lv >= 0) & (lv < _NODES_PER_TILE)
            els = plsc.load_gather(el2v, [srcv])
            erd = plsc.load_gather(er2v, [dstv])
            t = els + erd
            ee = jnp.exp(_lrelu(t) - cv)
            plsc.store_compressed(msrc.at[pl.ds(mcount, 16)], srcv, mask=m)
            plsc.store_compressed(mloc.at[pl.ds(mcount, 16)], lv, mask=m)
            plsc.store_compressed(mee.at[pl.ds(mcount, 16)], ee, mask=m)
            cntv = plsc.all_reduce_population_count(m)
            return mcount + cntv[0]

        mcount = lax.fori_loop(0, _CH // 16, scan_body, 0)
        msrc[pl.ds(mcount, 16)] = lane
        mloc[pl.ds(mcount, 16)] = jnp.full((16,), _NODES_PER_TILE,
                                           jnp.int32)
        mee[pl.ds(mcount, 16)] = z16
        ngroups = (mcount + 15) // 16

        def group_body(g, _):
            srcv = msrc[pl.ds(g * 16, 16)]
            locv = mloc[pl.ds(g * 16, 16)]
            pltpu.async_copy(feat2.at[srcv], xbuf, sem1).wait()
            for j in range(16):
                aj = locv[j]
                ehv = plsc.load_gather(
                    mee, [jnp.full((16,), g * 16 + j, jnp.int32)])
                for vv in range(8):
                    xv = xbuf[j, pl.ds(vv * 16, 16)]
                    plsc.addupdate(acc.at[pl.ds(aj * 128 + vv * 16, 16)],
                                   ehv * xv)
                dv = jnp.where(lane == 0, ehv, 0.0)
                plsc.addupdate(den.at[pl.ds(aj * 16, 16)], dv)
            return 0

        lax.fori_loop(0, ngroups, group_body, 0)
        return 0

    lax.fori_loop(0, _E // _CH, chunk_body, 0)

    pltpu.sync_copy(acc.at[pl.ds(0, _NODES_PER_TILE * 128)],
                    z2.at[pl.ds(base * 128, _NODES_PER_TILE * 128)])
    pltpu.sync_copy(den.at[pl.ds(0, _NODES_PER_TILE * 16)],
                    den2.at[pl.ds(base * 16, _NODES_PER_TILE * 16)])


def _sc_layer2(feat2, el2, er2, pmax, src, dst):
    mesh = plsc.VectorSubcoreMesh(core_axis_name="c", subcore_axis_name="s")
    npt = _NODES_PER_TILE
    fn = functools.partial(
        pl.kernel, mesh=mesh,
        compiler_params=pltpu.CompilerParams(needs_layout_passes=False),
        out_type=[
            jax.ShapeDtypeStruct((_N * 128,), jnp.float32),
            jax.ShapeDtypeStruct((_N * 16,), jnp.float32),
        ],
        scratch_types=[
            pltpu.VMEM(((npt + 1) * 128,), jnp.float32),
            pltpu.VMEM(((npt + 1) * 16,), jnp.float32),
            pltpu.VMEM((_N,), jnp.float32),
            pltpu.VMEM((_N,), jnp.float32),
            pltpu.VMEM((_CH,), jnp.int32),
            pltpu.VMEM((_CH,), jnp.int32),
            pltpu.VMEM((_CH + 16,), jnp.int32),
            pltpu.VMEM((_CH + 16,), jnp.int32),
            pltpu.VMEM((_CH + 16,), jnp.float32),
            pltpu.VMEM((16, 128), jnp.float32),
            pltpu.VMEM((512,), jnp.float32),
            pltpu.SemaphoreType.DMA,
        ],
    )(_sc_l2_body)
    return fn(feat2, el2.reshape(_N), er2.reshape(_N),
              pmax.reshape(32 * 16), src, dst)


def _head_body(z2_ref, den2_ref, b2_ref, target_ref, dist_ref, atomW_ref,
               atomb_ref, protW_ref, protb_ref, distW_ref, distb_ref,
               d1W_ref, d1b_ref, d2W_ref, d2b_ref, outW_ref, outb_ref,
               out_ref):
    z2 = z2_ref[...]
    nb = z2.shape[0]
    den = den2_ref[...][:, :, 0:1]
    h2r = z2 / (den + 1e-9) + b2_ref[...].reshape(1, 1, _F2)
    h2r = jnp.where(h2r > 0, h2r, jnp.exp(jnp.minimum(h2r, 0.0)) - 1.0)
    atomW = atomW_ref[...]
    logits = jax.lax.dot_general(
        h2r.reshape(nb * _NPG, _F2), atomW,
        (((1,), (0,)), ((), ())), preferred_element_type=jnp.float32)
    w = jax.nn.sigmoid(logits + atomb_ref[0, 0]).reshape(nb, _NPG, 1)
    hsum = jnp.sum(h2r * w, axis=1)
    hmax = jnp.max(h2r, axis=1)

    x_prot = jax.lax.dot_general(
        target_ref[...], protW_ref[...],
        (((1,), (0,)), ((), ())), preferred_element_type=jnp.float32)
    x_prot = x_prot + protb_ref[...]
    x_dist = jax.lax.dot_general(
        dist_ref[...], distW_ref[...],
        (((1,), (0,)), ((), ())), preferred_element_type=jnp.float32)
    x_dist = x_dist + distb_ref[...]

    x = jnp.concatenate([x_prot, hsum, hmax, x_dist], axis=1)
    x = jax.lax.dot_general(x, d1W_ref[...],
                            (((1,), (0,)), ((), ())),
                            preferred_element_type=jnp.float32)
    x = jnp.maximum(x + d1b_ref[...], 0.0)
    x = jax.lax.dot_general(x, d2W_ref[...],
                            (((1,), (0,)), ((), ())),
                            preferred_element_type=jnp.float32)
    x = jnp.maximum(x + d2b_ref[...], 0.0)
    out = jax.lax.dot_general(x, outW_ref[...],
                              (((1,), (0,)), ((), ())),
                              preferred_element_type=jnp.float32)
    out_ref[...] = out + outb_ref[0, 0]


def _head(z2, den2, b2, target, dist, atomW, atomb, protW, protb, distW,
          distb, d1W, d1b, d2W, d2b, outW, outb):
    nblk = 4
    bb = _B // nblk
    cst = lambda shape: pl.BlockSpec(shape, lambda g: tuple(0 for _ in shape))
    out = pl.pallas_call(
        _head_body,
        grid=(nblk,),
        in_specs=[
            pl.BlockSpec((bb, _NPG, _F2), lambda g: (g, 0, 0)),
            pl.BlockSpec((bb, _NPG, 16), lambda g: (g, 0, 0)),
            cst((1, _F2)),
            pl.BlockSpec((bb, 1024), lambda g: (g, 0)),
            pl.BlockSpec((bb, 10000), lambda g: (g, 0)),
            cst((_F2, 1)), cst((1, 1)),
            cst((1024, 256)), cst((1, 256)),
            cst((10000, 256)), cst((1, 256)),
            cst((768, 1024)), cst((1, 1024)),
            cst((1024, 256)), cst((1, 256)),
            cst((256, 1)), cst((1, 1)),
        ],
        out_specs=pl.BlockSpec((bb, 1), lambda g: (g, 0)),
        out_shape=jax.ShapeDtypeStruct((_B, 1), jnp.float32),
    )(z2.reshape(_B, _NPG, _F2), den2.reshape(_B, _NPG, 16),
      b2.reshape(1, _F2), target, dist, atomW, atomb.reshape(1, 1),
      protW, protb.reshape(1, -1), distW, distb.reshape(1, -1),
      d1W, d1b.reshape(1, -1), d2W, d2b.reshape(1, -1),
      outW, outb.reshape(1, 1))
    return out[:, 0]


def _gat_layer1(x, src, dst, W, al, ar, b):
    n = x.shape[0]
    feat = (x @ W).reshape(n, _H1, _F1)
    el = jnp.sum(feat * al[None, :, :], axis=-1)
    er = jnp.sum(feat * ar[None, :, :], axis=-1)
    c = _lrelu(jnp.max(el) + jnp.max(er))
    e = _lrelu(el[src] + er[dst])
    ee = jnp.exp(e - c)
    denom = jax.ops.segment_sum(ee, dst, num_segments=n)
    num = jax.ops.segment_sum(feat[src] * ee[:, :, None], dst,
                              num_segments=n)
    out = num / (denom[:, :, None] + 1e-9)
    out = out + b.reshape(1, _H1, _F1)
    return jax.nn.elu(out.reshape(n, _H1 * _F1))


def kernel(node_feats, edge_index, node_graph_ids, target, dist, W1, al1,
           ar1, b1, W2, al2, ar2, b2, atomW, atomb, protW, protb, distW,
           distb, d1W, d1b, d2W, d2b, outW, outb):
    src, dst = edge_index[0], edge_index[1]
    h = _gat_layer1(node_feats, src, dst, W1, al1, ar1, b1)
    feat2, el2, er2, pmax = _proj2(h, W2, al2, ar2)
    z2, den2 = _sc_layer2(feat2, el2, er2, pmax, src, dst)
    return _head(z2, den2, b2, target, dist, atomW, atomb, protW, protb,
                 distW, distb, d1W, d1b, d2W, d2b, outW, outb)


# speedup vs baseline: 9.8324x; 7.6501x over previous
import functools

import jax
import jax.numpy as jnp
from jax import lax
from jax.experimental import pallas as pl
from jax.experimental.pallas import tpu as pltpu
from jax.experimental.pallas import tpu_sc as plsc

_B = 512
_NPG = 32
_N = _B * _NPG
_E = _N * 3
_H1, _F1 = 10, 74
_F2 = 128
_CH = 2048
_NW = 32
_NODES_PER_TILE = _N // _NW


def _lrelu(x):
    return jnp.maximum(x, 0.2 * x)


def _proj2_body(h_ref, W2_ref, al2_ref, ar2_ref, feat2_ref, el2_ref,
                er2_ref, pmax_ref):
    feat2 = jax.lax.dot_general(
        h_ref[...], W2_ref[...], (((1,), (0,)), ((), ())),
        preferred_element_type=jnp.float32)
    feat2_ref[...] = feat2
    el2 = jax.lax.dot_general(
        feat2, al2_ref[...], (((1,), (1,)), ((), ())),
        preferred_element_type=jnp.float32)
    er2 = jax.lax.dot_general(
        feat2, ar2_ref[...], (((1,), (1,)), ((), ())),
        preferred_element_type=jnp.float32)
    el2_ref[...] = el2
    er2_ref[...] = er2
    lane = lax.broadcasted_iota(jnp.int32, (1, 1, 16), 2)
    mel = jnp.max(el2)
    mer = jnp.max(er2)
    pmax_ref[...] = jnp.where(lane == 0, mel,
                              jnp.where(lane == 1, mer, -1e30))


def _proj2(h, W2, al2, ar2):
    nblk = 32
    blk = _N // nblk
    return pl.pallas_call(
        _proj2_body,
        grid=(nblk,),
        in_specs=[
            pl.BlockSpec((blk, _H1 * _F1), lambda g: (g, 0)),
            pl.BlockSpec((_H1 * _F1, _F2), lambda g: (0, 0)),
            pl.BlockSpec((1, _F2), lambda g: (0, 0)),
            pl.BlockSpec((1, _F2), lambda g: (0, 0)),
        ],
        out_specs=[
            pl.BlockSpec((blk, _F2), lambda g: (g, 0)),
            pl.BlockSpec((blk, 1), lambda g: (g, 0)),
            pl.BlockSpec((blk, 1), lambda g: (g, 0)),
            pl.BlockSpec((1, 1, 16), lambda g: (g, 0, 0)),
        ],
        out_shape=[
            jax.ShapeDtypeStruct((_N, _F2), jnp.float32),
            jax.ShapeDtypeStruct((_N, 1), jnp.float32),
            jax.ShapeDtypeStruct((_N, 1), jnp.float32),
            jax.ShapeDtypeStruct((nblk, 1, 16), jnp.float32),
        ],
    )(h, W2, al2, ar2)


def _sc_l2_body(feat2, el2h, er2h, pmaxh, srch, dsth,
                z2, den2,
                acc, den, el2v, er2v, dstc, srcc, msrc, mloc, mee,
                xbuf, pmaxb, sem1):
    cc = lax.axis_index("c")
    ss = lax.axis_index("s")
    w = ss * 2 + cc
    base = w * _NODES_PER_TILE
    lane = lax.broadcasted_iota(jnp.int32, (16,), 0)
    z16 = jnp.zeros((16,), jnp.float32)

    pltpu.sync_copy(el2h, el2v)
    pltpu.sync_copy(er2h, er2v)
    pltpu.sync_copy(pmaxh, pmaxb)

    def _pm(i, mv):
        return jnp.maximum(mv, pmaxb[pl.ds(i * 16, 16)])
    mv = lax.fori_loop(1, 32, _pm, pmaxb[pl.ds(0, 16)])
    cv = jnp.full((16,), _lrelu(mv[0] + mv[1]), jnp.float32)

    def _zero(i, _):
        for k in range(8):
            acc[pl.ds(i * 128 + k * 16, 16)] = z16
        den[pl.ds(i * 16, 16)] = z16
        return 0
    lax.fori_loop(0, _NODES_PER_TILE + 1, _zero, 0)

    def chunk_body(ch, _):
        pltpu.sync_copy(dsth.at[pl.ds(ch * _CH, _CH)], dstc)
        pltpu.sync_copy(srch.at[pl.ds(ch * _CH, _CH)], srcc)

        def scan_body(v, mcount):
            dstv = dstc[pl.ds(v * 16, 16)]
            srcv = srcc[pl.ds(v * 16, 16)]
            lv = dstv - base
            m = (lv >= 0) & (lv < _NODES_PER_TILE)
            els = plsc.load_gather(el2v, [srcv])
            erd = plsc.load_gather(er2v, [dstv])
            t = els + erd
            ee = jnp.exp(_lrelu(t) - cv)
            plsc.store_compressed(msrc.at[pl.ds(mcount, 16)], srcv, mask=m)
            plsc.store_compressed(mloc.at[pl.ds(mcount, 16)], lv, mask=m)
            plsc.store_compressed(mee.at[pl.ds(mcount, 16)], ee, mask=m)
            cntv = plsc.all_reduce_population_count(m)
            return mcount + cntv[0]

        mcount = lax.fori_loop(0, _CH // 16, scan_body, 0)
        msrc[pl.ds(mcount, 16)] = lane
        mloc[pl.ds(mcount, 16)] = jnp.full((16,), _NODES_PER_TILE,
                                           jnp.int32)
        mee[pl.ds(mcount, 16)] = z16
        ngroups = (mcount + 15) // 16

        def group_body(g, _):
            srcv = msrc[pl.ds(g * 16, 16)]
            locv = mloc[pl.ds(g * 16, 16)]
            pltpu.async_copy(feat2.at[srcv], xbuf, sem1).wait()
            for j in range(16):
                aj = locv[j]
                ehv = plsc.load_gather(
                    mee, [jnp.full((16,), g * 16 + j, jnp.int32)])
                for vv in range(8):
                    xv = xbuf[j, pl.ds(vv * 16, 16)]
                    plsc.addupdate(acc.at[pl.ds(aj * 128 + vv * 16, 16)],
                                   ehv * xv)
                dv = jnp.where(lane == 0, ehv, 0.0)
                plsc.addupdate(den.at[pl.ds(aj * 16, 16)], dv)
            return 0

        lax.fori_loop(0, ngroups, group_body, 0)
        return 0

    lax.fori_loop(0, _E // _CH, chunk_body, 0)

    pltpu.sync_copy(acc.at[pl.ds(0, _NODES_PER_TILE * 128)],
                    z2.at[pl.ds(base * 128, _NODES_PER_TILE * 128)])
    pltpu.sync_copy(den.at[pl.ds(0, _NODES_PER_TILE * 16)],
                    den2.at[pl.ds(base * 16, _NODES_PER_TILE * 16)])


def _sc_layer2(feat2, el2, er2, pmax, src, dst):
    mesh = plsc.VectorSubcoreMesh(core_axis_name="c", subcore_axis_name="s")
    npt = _NODES_PER_TILE
    fn = functools.partial(
        pl.kernel, mesh=mesh,
        compiler_params=pltpu.CompilerParams(needs_layout_passes=False),
        out_type=[
            jax.ShapeDtypeStruct((_N * 128,), jnp.float32),
            jax.ShapeDtypeStruct((_N * 16,), jnp.float32),
        ],
        scratch_types=[
            pltpu.VMEM(((npt + 1) * 128,), jnp.float32),
            pltpu.VMEM(((npt + 1) * 16,), jnp.float32),
            pltpu.VMEM((_N,), jnp.float32),
            pltpu.VMEM((_N,), jnp.float32),
            pltpu.VMEM((_CH,), jnp.int32),
            pltpu.VMEM((_CH,), jnp.int32),
            pltpu.VMEM((_CH + 16,), jnp.int32),
            pltpu.VMEM((_CH + 16,), jnp.int32),
            pltpu.VMEM((_CH + 16,), jnp.float32),
            pltpu.VMEM((16, 128), jnp.float32),
            pltpu.VMEM((512,), jnp.float32),
            pltpu.SemaphoreType.DMA,
        ],
    )(_sc_l2_body)
    return fn(feat2, el2.reshape(_N), er2.reshape(_N),
              pmax.reshape(32 * 16), src, dst)


def _head_body(z2_ref, den2_ref, b2_ref, target_ref, dist_ref, atomW_ref,
               atomb_ref, protW_ref, protb_ref, distW_ref, distb_ref,
               d1W_ref, d1b_ref, d2W_ref, d2b_ref, outW_ref, outb_ref,
               out_ref):
    z2 = z2_ref[...]
    nb = z2.shape[0]
    den = den2_ref[...][:, :, 0:1]
    h2r = z2 / (den + 1e-9) + b2_ref[...].reshape(1, 1, _F2)
    h2r = jnp.where(h2r > 0, h2r, jnp.exp(jnp.minimum(h2r, 0.0)) - 1.0)
    atomW = atomW_ref[...]
    logits = jax.lax.dot_general(
        h2r.reshape(nb * _NPG, _F2), atomW,
        (((1,), (0,)), ((), ())), preferred_element_type=jnp.float32)
    w = jax.nn.sigmoid(logits + atomb_ref[0, 0]).reshape(nb, _NPG, 1)
    hsum = jnp.sum(h2r * w, axis=1)
    hmax = jnp.max(h2r, axis=1)

    x_prot = jax.lax.dot_general(
        target_ref[...], protW_ref[...],
        (((1,), (0,)), ((), ())), preferred_element_type=jnp.float32)
    x_prot = x_prot + protb_ref[...]
    x_dist = jax.lax.dot_general(
        dist_ref[...], distW_ref[...],
        (((1,), (0,)), ((), ())), preferred_element_type=jnp.float32)
    x_dist = x_dist + distb_ref[...]

    x = jnp.concatenate([x_prot, hsum, hmax, x_dist], axis=1)
    x = jax.lax.dot_general(x, d1W_ref[...],
                            (((1,), (0,)), ((), ())),
                            preferred_element_type=jnp.float32)
    x = jnp.maximum(x + d1b_ref[...], 0.0)
    x = jax.lax.dot_general(x, d2W_ref[...],
                            (((1,), (0,)), ((), ())),
                            preferred_element_type=jnp.float32)
    x = jnp.maximum(x + d2b_ref[...], 0.0)
    out = jax.lax.dot_general(x, outW_ref[...],
                              (((1,), (0,)), ((), ())),
                              preferred_element_type=jnp.float32)
    out_ref[...] = out + outb_ref[0, 0]


def _head(z2, den2, b2, target, dist, atomW, atomb, protW, protb, distW,
          distb, d1W, d1b, d2W, d2b, outW, outb):
    nblk = 4
    bb = _B // nblk
    cst = lambda shape: pl.BlockSpec(shape, lambda g: tuple(0 for _ in shape))
    out = pl.pallas_call(
        _head_body,
        grid=(nblk,),
        in_specs=[
            pl.BlockSpec((bb, _NPG, _F2), lambda g: (g, 0, 0)),
            pl.BlockSpec((bb, _NPG, 16), lambda g: (g, 0, 0)),
            cst((1, _F2)),
            pl.BlockSpec((bb, 1024), lambda g: (g, 0)),
            pl.BlockSpec((bb, 10000), lambda g: (g, 0)),
            cst((_F2, 1)), cst((1, 1)),
            cst((1024, 256)), cst((1, 256)),
            cst((10000, 256)), cst((1, 256)),
            cst((768, 1024)), cst((1, 1024)),
            cst((1024, 256)), cst((1, 256)),
            cst((256, 1)), cst((1, 1)),
        ],
        out_specs=pl.BlockSpec((bb, 1), lambda g: (g, 0)),
        out_shape=jax.ShapeDtypeStruct((_B, 1), jnp.float32),
    )(z2.reshape(_B, _NPG, _F2), den2.reshape(_B, _NPG, 16),
      b2.reshape(1, _F2), target, dist, atomW, atomb.reshape(1, 1),
      protW, protb.reshape(1, -1), distW, distb.reshape(1, -1),
      d1W, d1b.reshape(1, -1), d2W, d2b.reshape(1, -1),
      outW, outb.reshape(1, 1))
    return out[:, 0]


def _attn1_body(x_ref, Wel_ref, Wer_ref, packed_ref, pmax_ref):
    el = jax.lax.dot_general(
        x_ref[...], Wel_ref[...], (((1,), (0,)), ((), ())),
        preferred_element_type=jnp.float32)
    er = jax.lax.dot_general(
        x_ref[...], Wer_ref[...], (((1,), (0,)), ((), ())),
        preferred_element_type=jnp.float32)
    blk = el.shape[0]
    fill = jnp.full((blk, 16 - _H1), -1e30, jnp.float32)
    zpad6 = jnp.zeros((blk, 6), jnp.float32)
    zpad16 = jnp.zeros((blk, 16), jnp.float32)
    packed_ref[...] = jnp.concatenate(
        [x_ref[...], zpad6, el, fill, er, fill, zpad16], axis=1)
    lane = lax.broadcasted_iota(jnp.int32, (1, 1, 16), 2)
    mel = jnp.max(el)
    mer = jnp.max(er)
    pmax_ref[...] = jnp.where(lane == 0, mel,
                              jnp.where(lane == 1, mer, -1e30))


def _attn1(x, Wel, Wer):
    nblk = 32
    blk = _N // nblk
    return pl.pallas_call(
        _attn1_body,
        grid=(nblk,),
        in_specs=[
            pl.BlockSpec((blk, 74), lambda g: (g, 0)),
            pl.BlockSpec((74, _H1), lambda g: (0, 0)),
            pl.BlockSpec((74, _H1), lambda g: (0, 0)),
        ],
        out_specs=[
            pl.BlockSpec((blk, 128), lambda g: (g, 0)),
            pl.BlockSpec((1, 1, 16), lambda g: (g, 0, 0)),
        ],
        out_shape=[
            jax.ShapeDtypeStruct((_N, 128), jnp.float32),
            jax.ShapeDtypeStruct((nblk, 1, 16), jnp.float32),
        ],
    )(x, Wel, Wer)


def _sc_l1_body(xh, pmaxh, srch, dsth,
                z1, den1,
                acc, den, dstc, srcc, msrc, mloc,
                sbuf, dbuf, pmaxb, sem1, sem2):
    cc = lax.axis_index("c")
    ss = lax.axis_index("s")
    w = ss * 2 + cc
    lane = lax.broadcasted_iota(jnp.int32, (16,), 0)
    z16 = jnp.zeros((16,), jnp.float32)
    hm = (lane < _H1)

    pltpu.sync_copy(pmaxh, pmaxb)

    def _pm(i, mv):
        return jnp.maximum(mv, pmaxb[pl.ds(i * 16, 16)])
    mv = lax.fori_loop(1, 32, _pm, pmaxb[pl.ds(0, 16)])
    cv = jnp.full((16,), _lrelu(mv[0] + mv[1]), jnp.float32)

    def pass_body(p, _):
        base = (p * 32 + w) * 128

        def _zero(i, _):
            for k in range(5):
                acc[pl.ds(i * 80 + k * 16, 16)] = z16
            return 0
        lax.fori_loop(0, 129 * 10, _zero, 0)

        def _zeroden(i, _):
            den[pl.ds(i * 16, 16)] = z16
            return 0
        lax.fori_loop(0, 129, _zeroden, 0)

        def chunk_body(ch, _):
            pltpu.sync_copy(dsth.at[pl.ds(ch * _CH, _CH)], dstc)
            pltpu.sync_copy(srch.at[pl.ds(ch * _CH, _CH)], srcc)

            def scan_body(v, mcount):
                dstv = dstc[pl.ds(v * 16, 16)]
                srcv = srcc[pl.ds(v * 16, 16)]
                lv = dstv - base
                m = (lv >= 0) & (lv < 128)
                plsc.store_compressed(msrc.at[pl.ds(mcount, 16)], srcv,
                                      mask=m)
                plsc.store_compressed(mloc.at[pl.ds(mcount, 16)], lv,
                                      mask=m)
                cntv = plsc.all_reduce_population_count(m)
                return mcount + cntv[0]

            mcount = lax.fori_loop(0, _CH // 16, scan_body, 0)
            msrc[pl.ds(mcount, 16)] = lane
            mloc[pl.ds(mcount, 16)] = jnp.full((16,), 128, jnp.int32)
            ngroups = (mcount + 15) // 16

            def group_body(g, _):
                srcv = msrc[pl.ds(g * 16, 16)]
                locv = mloc[pl.ds(g * 16, 16)]
                dstg = jnp.minimum(locv, 127) + base
                c1 = pltpu.async_copy(xh.at[srcv], sbuf, sem1)
                c2 = pltpu.async_copy(xh.at[dstg], dbuf, sem2)
                c1.wait()
                c2.wait()
                for j in range(16):
                    aj = locv[j]
                    t = sbuf[j, pl.ds(80, 16)] + dbuf[j, pl.ds(96, 16)]
                    eej = jnp.where(hm, jnp.exp(_lrelu(t) - cv), 0.0)
                    plsc.addupdate(den.at[pl.ds(aj * 16, 16)], eej)
                    xv = [sbuf[j, pl.ds(vv * 16, 16)] for vv in range(5)]
                    for h in range(_H1):
                        ehv = jnp.full((16,), eej[h], jnp.float32)
                        for vv in range(5):
                            plsc.addupdate(
                                acc.at[pl.ds(aj * 800 + h * 80 + vv * 16,
                                             16)],
                                ehv * xv[vv])
                return 0

            lax.fori_loop(0, ngroups, group_body, 0)
            return 0

        lax.fori_loop(0, _E // _CH, chunk_body, 0)

        pltpu.sync_copy(acc.at[pl.ds(0, 128 * 800)],
                        z1.at[pl.ds(base * 800, 128 * 800)])
        pltpu.sync_copy(den.at[pl.ds(0, 128 * 16)],
                        den1.at[pl.ds(base * 16, 128 * 16)])
        return 0

    lax.fori_loop(0, 4, pass_body, 0)


def _sc_layer1(x_packed, pmax, src, dst):
    mesh = plsc.VectorSubcoreMesh(core_axis_name="c", subcore_axis_name="s")
    fn = functools.partial(
        pl.kernel, mesh=mesh,
        compiler_params=pltpu.CompilerParams(needs_layout_passes=False),
        out_type=[
            jax.ShapeDtypeStruct((_N * 800,), jnp.float32),
            jax.ShapeDtypeStruct((_N * 16,), jnp.float32),
        ],
        scratch_types=[
            pltpu.VMEM((129 * 800,), jnp.float32),
            pltpu.VMEM((129 * 16,), jnp.float32),
            pltpu.VMEM((_CH,), jnp.int32),
            pltpu.VMEM((_CH,), jnp.int32),
            pltpu.VMEM((_CH + 16,), jnp.int32),
            pltpu.VMEM((_CH + 16,), jnp.int32),
            pltpu.VMEM((16, 128), jnp.float32),
            pltpu.VMEM((16, 128), jnp.float32),
            pltpu.VMEM((512,), jnp.float32),
            pltpu.SemaphoreType.DMA,
            pltpu.SemaphoreType.DMA,
        ],
    )(_sc_l1_body)
    return fn(x_packed, pmax.reshape(32 * 16), src, dst)


def _agg1_body(z1_ref, den1_ref, b1_ref, W1h_ref, W2_ref, al2_ref,
               ar2_ref, feat2_ref, el2_ref, er2_ref, pmax_ref):
    blk = z1_ref.shape[0]
    den1 = den1_ref[...]
    feat2 = jnp.zeros((blk, _F2), jnp.float32)
    for h in range(_H1):
        zh = z1_ref[:, pl.ds(h * 80, 74)]
        oh = jax.lax.dot_general(
            zh, W1h_ref[h], (((1,), (0,)), ((), ())),
            preferred_element_type=jnp.float32)
        oh = oh / (den1[:, h:h + 1] + 1e-9)
        oh = oh + b1_ref[0, pl.ds(h * _F1, _F1)].reshape(1, _F1)
        h1h = jnp.where(oh > 0, oh, jnp.exp(jnp.minimum(oh, 0.0)) - 1.0)
        feat2 = feat2 + jax.lax.dot_general(
            h1h, W2_ref[pl.ds(h * _F1, _F1), :],
            (((1,), (0,)), ((), ())), preferred_element_type=jnp.float32)
    feat2_ref[...] = feat2
    el2 = jax.lax.dot_general(
        feat2, al2_ref[...], (((1,), (1,)), ((), ())),
        preferred_element_type=jnp.float32)
    er2 = jax.lax.dot_general(
        feat2, ar2_ref[...], (((1,), (1,)), ((), ())),
        preferred_element_type=jnp.float32)
    el2_ref[...] = el2
    er2_ref[...] = er2
    lane = lax.broadcasted_iota(jnp.int32, (1, 1, 16), 2)
    mel = jnp.max(el2)
    mer = jnp.max(er2)
    pmax_ref[...] = jnp.where(lane == 0, mel,
                              jnp.where(lane == 1, mer, -1e30))


def _agg1(z1, den1, b1, W1h, W2, al2, ar2):
    nblk = 32
    blk = _N // nblk
    return pl.pallas_call(
        _agg1_body,
        grid=(nblk,),
        in_specs=[
            pl.BlockSpec((blk, 800), lambda g: (g, 0)),
            pl.BlockSpec((blk, 16), lambda g: (g, 0)),
            pl.BlockSpec((1, _H1 * _F1), lambda g: (0, 0)),
            pl.BlockSpec((_H1, 74, 74), lambda g: (0, 0, 0)),
            pl.BlockSpec((_H1 * _F1, _F2), lambda g: (0, 0)),
            pl.BlockSpec((1, _F2), lambda g: (0, 0)),
            pl.BlockSpec((1, _F2), lambda g: (0, 0)),
        ],
        out_specs=[
            pl.BlockSpec((blk, _F2), lambda g: (g, 0)),
            pl.BlockSpec((blk, 1), lambda g: (g, 0)),
            pl.BlockSpec((blk, 1), lambda g: (g, 0)),
            pl.BlockSpec((1, 1, 16), lambda g: (g, 0, 0)),
        ],
        out_shape=[
            jax.ShapeDtypeStruct((_N, _F2), jnp.float32),
            jax.ShapeDtypeStruct((_N, 1), jnp.float32),
            jax.ShapeDtypeStruct((_N, 1), jnp.float32),
            jax.ShapeDtypeStruct((nblk, 1, 16), jnp.float32),
        ],
    )(z1.reshape(_N, 800), den1.reshape(_N, 16), b1.reshape(1, -1),
      W1h, W2, al2, ar2)


def kernel(node_feats, edge_index, node_graph_ids, target, dist, W1, al1,
           ar1, b1, W2, al2, ar2, b2, atomW, atomb, protW, protb, distW,
           distb, d1W, d1b, d2W, d2b, outW, outb):
    src, dst = edge_index[0], edge_index[1]
    W1r = W1.reshape(74, _H1, _F1)
    Wel1 = jnp.einsum('fhg,hg->fh', W1r, al1)
    Wer1 = jnp.einsum('fhg,hg->fh', W1r, ar1)
    W1h = W1r.transpose(1, 0, 2)
    x_packed, pmax1 = _attn1(node_feats, Wel1, Wer1)
    z1, den1 = _sc_layer1(x_packed, pmax1, src, dst)
    feat2, el2, er2, pmax = _agg1(z1, den1, b1, W1h, W2, al2, ar2)
    z2, den2 = _sc_layer2(feat2, el2, er2, pmax, src, dst)
    return _head(z2, den2, b2, target, dist, atomW, atomb, protW, protb,
                 distW, distb, d1W, d1b, d2W, d2b, outW, outb)


# speedup vs baseline: 11.0735x; 1.1262x over previous
import functools

import jax
import jax.numpy as jnp
from jax import lax
from jax.experimental import pallas as pl
from jax.experimental.pallas import tpu as pltpu
from jax.experimental.pallas import tpu_sc as plsc

_B = 512
_NPG = 32
_N = _B * _NPG
_E = _N * 3
_H1, _F1 = 10, 74
_F2 = 128
_CH = 4096
_NW = 32
_NODES_PER_TILE = _N // _NW


def _lrelu(x):
    return jnp.maximum(x, 0.2 * x)


def _proj2_body(h_ref, W2_ref, al2_ref, ar2_ref, feat2_ref, el2_ref,
                er2_ref, pmax_ref):
    feat2 = jax.lax.dot_general(
        h_ref[...], W2_ref[...], (((1,), (0,)), ((), ())),
        preferred_element_type=jnp.float32)
    feat2_ref[...] = feat2
    el2 = jax.lax.dot_general(
        feat2, al2_ref[...], (((1,), (1,)), ((), ())),
        preferred_element_type=jnp.float32)
    er2 = jax.lax.dot_general(
        feat2, ar2_ref[...], (((1,), (1,)), ((), ())),
        preferred_element_type=jnp.float32)
    el2_ref[...] = el2
    er2_ref[...] = er2
    lane = lax.broadcasted_iota(jnp.int32, (1, 1, 16), 2)
    mel = jnp.max(el2)
    mer = jnp.max(er2)
    pmax_ref[...] = jnp.where(lane == 0, mel,
                              jnp.where(lane == 1, mer, -1e30))


def _proj2(h, W2, al2, ar2):
    nblk = 32
    blk = _N // nblk
    return pl.pallas_call(
        _proj2_body,
        grid=(nblk,),
        in_specs=[
            pl.BlockSpec((blk, _H1 * _F1), lambda g: (g, 0)),
            pl.BlockSpec((_H1 * _F1, _F2), lambda g: (0, 0)),
            pl.BlockSpec((1, _F2), lambda g: (0, 0)),
            pl.BlockSpec((1, _F2), lambda g: (0, 0)),
        ],
        out_specs=[
            pl.BlockSpec((blk, _F2), lambda g: (g, 0)),
            pl.BlockSpec((blk, 1), lambda g: (g, 0)),
            pl.BlockSpec((blk, 1), lambda g: (g, 0)),
            pl.BlockSpec((1, 1, 16), lambda g: (g, 0, 0)),
        ],
        out_shape=[
            jax.ShapeDtypeStruct((_N, _F2), jnp.float32),
            jax.ShapeDtypeStruct((_N, 1), jnp.float32),
            jax.ShapeDtypeStruct((_N, 1), jnp.float32),
            jax.ShapeDtypeStruct((nblk, 1, 16), jnp.float32),
        ],
    )(h, W2, al2, ar2)


def _sc_l2_body(feat2, el2h, er2h, pmaxh, srch, dsth,
                z2, den2,
                acc, den, el2v, er2v, dstc, srcc, msrc, mloc, mee,
                xbuf, pmaxb, sem1):
    cc = lax.axis_index("c")
    ss = lax.axis_index("s")
    w = ss * 2 + cc
    base = w * _NODES_PER_TILE
    lane = lax.broadcasted_iota(jnp.int32, (16,), 0)
    z16 = jnp.zeros((16,), jnp.float32)

    pltpu.sync_copy(el2h, el2v)
    pltpu.sync_copy(er2h, er2v)
    pltpu.sync_copy(pmaxh, pmaxb)

    def _pm(i, mv):
        return jnp.maximum(mv, pmaxb[pl.ds(i * 16, 16)])
    mv = lax.fori_loop(1, 32, _pm, pmaxb[pl.ds(0, 16)])
    cv = jnp.full((16,), _lrelu(mv[0] + mv[1]), jnp.float32)

    def _zero(i, _):
        for k in range(8):
            acc[pl.ds(i * 128 + k * 16, 16)] = z16
        den[pl.ds(i * 16, 16)] = z16
        return 0
    lax.fori_loop(0, _NODES_PER_TILE + 1, _zero, 0)

    def chunk_body(ch, _):
        pltpu.sync_copy(dsth.at[pl.ds(ch * _CH, _CH)], dstc)
        pltpu.sync_copy(srch.at[pl.ds(ch * _CH, _CH)], srcc)

        def scan_body(v, mcount):
            dstv = dstc[pl.ds(v * 16, 16)]
            srcv = srcc[pl.ds(v * 16, 16)]
            lv = dstv - base
            m = (lv >= 0) & (lv < _NODES_PER_TILE)
            els = plsc.load_gather(el2v, [srcv])
            erd = plsc.load_gather(er2v, [dstv])
            t = els + erd
            ee = jnp.exp(_lrelu(t) - cv)
            plsc.store_compressed(msrc.at[pl.ds(mcount, 16)], srcv, mask=m)
            plsc.store_compressed(mloc.at[pl.ds(mcount, 16)], lv, mask=m)
            plsc.store_compressed(mee.at[pl.ds(mcount, 16)], ee, mask=m)
            cntv = plsc.all_reduce_population_count(m)
            return mcount + cntv[0]

        mcount = lax.fori_loop(0, _CH // 16, scan_body, 0)
        msrc[pl.ds(mcount, 16)] = lane
        mloc[pl.ds(mcount, 16)] = jnp.full((16,), _NODES_PER_TILE,
                                           jnp.int32)
        mee[pl.ds(mcount, 16)] = z16
        ngroups = (mcount + 15) // 16

        def group_body(g, _):
            srcv = msrc[pl.ds(g * 16, 16)]
            locv = mloc[pl.ds(g * 16, 16)]
            pltpu.async_copy(feat2.at[srcv], xbuf, sem1).wait()
            for j in range(16):
                aj = locv[j]
                ehv = plsc.load_gather(
                    mee, [jnp.full((16,), g * 16 + j, jnp.int32)])
                for vv in range(8):
                    xv = xbuf[j, pl.ds(vv * 16, 16)]
                    plsc.addupdate(acc.at[pl.ds(aj * 128 + vv * 16, 16)],
                                   ehv * xv)
                dv = jnp.where(lane == 0, ehv, 0.0)
                plsc.addupdate(den.at[pl.ds(aj * 16, 16)], dv)
            return 0

        lax.fori_loop(0, ngroups, group_body, 0)
        return 0

    lax.fori_loop(0, _E // _CH, chunk_body, 0)

    pltpu.sync_copy(acc.at[pl.ds(0, _NODES_PER_TILE * 128)],
                    z2.at[pl.ds(base * 128, _NODES_PER_TILE * 128)])
    pltpu.sync_copy(den.at[pl.ds(0, _NODES_PER_TILE * 16)],
                    den2.at[pl.ds(base * 16, _NODES_PER_TILE * 16)])


def _sc_layer2(feat2, el2, er2, pmax, src, dst):
    mesh = plsc.VectorSubcoreMesh(core_axis_name="c", subcore_axis_name="s")
    npt = _NODES_PER_TILE
    fn = functools.partial(
        pl.kernel, mesh=mesh,
        compiler_params=pltpu.CompilerParams(needs_layout_passes=False),
        out_type=[
            jax.ShapeDtypeStruct((_N * 128,), jnp.float32),
            jax.ShapeDtypeStruct((_N * 16,), jnp.float32),
        ],
        scratch_types=[
            pltpu.VMEM(((npt + 1) * 128,), jnp.float32),
            pltpu.VMEM(((npt + 1) * 16,), jnp.float32),
            pltpu.VMEM((_N,), jnp.float32),
            pltpu.VMEM((_N,), jnp.float32),
            pltpu.VMEM((_CH,), jnp.int32),
            pltpu.VMEM((_CH,), jnp.int32),
            pltpu.VMEM((_CH + 16,), jnp.int32),
            pltpu.VMEM((_CH + 16,), jnp.int32),
            pltpu.VMEM((_CH + 16,), jnp.float32),
            pltpu.VMEM((16, 128), jnp.float32),
            pltpu.VMEM((512,), jnp.float32),
            pltpu.SemaphoreType.DMA,
        ],
    )(_sc_l2_body)
    return fn(feat2, el2.reshape(_N), er2.reshape(_N),
              pmax.reshape(32 * 16), src, dst)


def _head_body(z2_ref, den2_ref, b2_ref, target_ref, dist_ref, atomW_ref,
               atomb_ref, protW_ref, protb_ref, distW_ref, distb_ref,
               d1W_ref, d1b_ref, d2W_ref, d2b_ref, outW_ref, outb_ref,
               out_ref):
    z2 = z2_ref[...]
    nb = z2.shape[0]
    den = den2_ref[...][:, :, 0:1]
    h2r = z2 / (den + 1e-9) + b2_ref[...].reshape(1, 1, _F2)
    h2r = jnp.where(h2r > 0, h2r, jnp.exp(jnp.minimum(h2r, 0.0)) - 1.0)
    atomW = atomW_ref[...]
    logits = jax.lax.dot_general(
        h2r.reshape(nb * _NPG, _F2), atomW,
        (((1,), (0,)), ((), ())), preferred_element_type=jnp.float32)
    w = jax.nn.sigmoid(logits + atomb_ref[0, 0]).reshape(nb, _NPG, 1)
    hsum = jnp.sum(h2r * w, axis=1)
    hmax = jnp.max(h2r, axis=1)

    x_prot = jax.lax.dot_general(
        target_ref[...], protW_ref[...],
        (((1,), (0,)), ((), ())), preferred_element_type=jnp.float32)
    x_prot = x_prot + protb_ref[...]
    x_dist = jax.lax.dot_general(
        dist_ref[...], distW_ref[...],
        (((1,), (0,)), ((), ())), preferred_element_type=jnp.float32)
    x_dist = x_dist + distb_ref[...]

    x = jnp.concatenate([x_prot, hsum, hmax, x_dist], axis=1)
    x = jax.lax.dot_general(x, d1W_ref[...],
                            (((1,), (0,)), ((), ())),
                            preferred_element_type=jnp.float32)
    x = jnp.maximum(x + d1b_ref[...], 0.0)
    x = jax.lax.dot_general(x, d2W_ref[...],
                            (((1,), (0,)), ((), ())),
                            preferred_element_type=jnp.float32)
    x = jnp.maximum(x + d2b_ref[...], 0.0)
    out = jax.lax.dot_general(x, outW_ref[...],
                              (((1,), (0,)), ((), ())),
                              preferred_element_type=jnp.float32)
    out_ref[...] = out + outb_ref[0, 0]


def _head(z2, den2, b2, target, dist, atomW, atomb, protW, protb, distW,
          distb, d1W, d1b, d2W, d2b, outW, outb):
    nblk = 4
    bb = _B // nblk
    cst = lambda shape: pl.BlockSpec(shape, lambda g: tuple(0 for _ in shape))
    out = pl.pallas_call(
        _head_body,
        grid=(nblk,),
        in_specs=[
            pl.BlockSpec((bb, _NPG, _F2), lambda g: (g, 0, 0)),
            pl.BlockSpec((bb, _NPG, 16), lambda g: (g, 0, 0)),
            cst((1, _F2)),
            pl.BlockSpec((bb, 1024), lambda g: (g, 0)),
            pl.BlockSpec((bb, 10000), lambda g: (g, 0)),
            cst((_F2, 1)), cst((1, 1)),
            cst((1024, 256)), cst((1, 256)),
            cst((10000, 256)), cst((1, 256)),
            cst((768, 1024)), cst((1, 1024)),
            cst((1024, 256)), cst((1, 256)),
            cst((256, 1)), cst((1, 1)),
        ],
        out_specs=pl.BlockSpec((bb, 1), lambda g: (g, 0)),
        out_shape=jax.ShapeDtypeStruct((_B, 1), jnp.float32),
    )(z2.reshape(_B, _NPG, _F2), den2.reshape(_B, _NPG, 16),
      b2.reshape(1, _F2), target, dist, atomW, atomb.reshape(1, 1),
      protW, protb.reshape(1, -1), distW, distb.reshape(1, -1),
      d1W, d1b.reshape(1, -1), d2W, d2b.reshape(1, -1),
      outW, outb.reshape(1, 1))
    return out[:, 0]


def _attn1_body(x_ref, Wel_ref, Wer_ref, packed_ref, pmax_ref):
    el = jax.lax.dot_general(
        x_ref[...], Wel_ref[...], (((1,), (0,)), ((), ())),
        preferred_element_type=jnp.float32)
    er = jax.lax.dot_general(
        x_ref[...], Wer_ref[...], (((1,), (0,)), ((), ())),
        preferred_element_type=jnp.float32)
    blk = el.shape[0]
    fill = jnp.full((blk, 16 - _H1), -1e30, jnp.float32)
    zpad6 = jnp.zeros((blk, 6), jnp.float32)
    zpad16 = jnp.zeros((blk, 16), jnp.float32)
    packed_ref[...] = jnp.concatenate(
        [x_ref[...], zpad6, el, fill, er, fill, zpad16], axis=1)
    lane = lax.broadcasted_iota(jnp.int32, (1, 1, 16), 2)
    mel = jnp.max(el)
    mer = jnp.max(er)
    pmax_ref[...] = jnp.where(lane == 0, mel,
                              jnp.where(lane == 1, mer, -1e30))


def _attn1(x, Wel, Wer):
    nblk = 32
    blk = _N // nblk
    return pl.pallas_call(
        _attn1_body,
        grid=(nblk,),
        in_specs=[
            pl.BlockSpec((blk, 74), lambda g: (g, 0)),
            pl.BlockSpec((74, _H1), lambda g: (0, 0)),
            pl.BlockSpec((74, _H1), lambda g: (0, 0)),
        ],
        out_specs=[
            pl.BlockSpec((blk, 128), lambda g: (g, 0)),
            pl.BlockSpec((1, 1, 16), lambda g: (g, 0, 0)),
        ],
        out_shape=[
            jax.ShapeDtypeStruct((_N, 128), jnp.float32),
            jax.ShapeDtypeStruct((nblk, 1, 16), jnp.float32),
        ],
    )(x, Wel, Wer)


def _sc_l1_body(xh, pmaxh, srch, dsth,
                z1, den1,
                acc, den, dstc, srcc, msrc, mloc,
                sbuf, dbuf, pmaxb, sem1, sem2):
    cc = lax.axis_index("c")
    ss = lax.axis_index("s")
    w = ss * 2 + cc
    lane = lax.broadcasted_iota(jnp.int32, (16,), 0)
    z16 = jnp.zeros((16,), jnp.float32)
    hm = (lane < _H1)

    pltpu.sync_copy(pmaxh, pmaxb)

    def _pm(i, mv):
        return jnp.maximum(mv, pmaxb[pl.ds(i * 16, 16)])
    mv = lax.fori_loop(1, 32, _pm, pmaxb[pl.ds(0, 16)])
    cv = jnp.full((16,), _lrelu(mv[0] + mv[1]), jnp.float32)

    def pass_body(p, _):
        base = (p * 32 + w) * 128

        def _zero(i, _):
            for k in range(5):
                acc[pl.ds(i * 80 + k * 16, 16)] = z16
            return 0
        lax.fori_loop(0, 129 * 10, _zero, 0)

        def _zeroden(i, _):
            den[pl.ds(i * 16, 16)] = z16
            return 0
        lax.fori_loop(0, 129, _zeroden, 0)

        def chunk_body(ch, _):
            pltpu.sync_copy(dsth.at[pl.ds(ch * _CH, _CH)], dstc)
            pltpu.sync_copy(srch.at[pl.ds(ch * _CH, _CH)], srcc)

            def scan_body(v, mcount):
                dstv = dstc[pl.ds(v * 16, 16)]
                srcv = srcc[pl.ds(v * 16, 16)]
                lv = dstv - base
                m = (lv >= 0) & (lv < 128)
                plsc.store_compressed(msrc.at[pl.ds(mcount, 16)], srcv,
                                      mask=m)
                plsc.store_compressed(mloc.at[pl.ds(mcount, 16)], lv,
                                      mask=m)
                cntv = plsc.all_reduce_population_count(m)
                return mcount + cntv[0]

            mcount = lax.fori_loop(0, _CH // 16, scan_body, 0)
            msrc[pl.ds(mcount, 16)] = lane
            mloc[pl.ds(mcount, 16)] = jnp.full((16,), 128, jnp.int32)
            ngroups = (mcount + 15) // 16

            def group_body(g, _):
                srcv = msrc[pl.ds(g * 16, 16)]
                locv = mloc[pl.ds(g * 16, 16)]
                dstg = jnp.minimum(locv, 127) + base
                c1 = pltpu.async_copy(xh.at[srcv], sbuf, sem1)
                c2 = pltpu.async_copy(xh.at[dstg], dbuf, sem2)
                c1.wait()
                c2.wait()
                for j in range(16):
                    aj = locv[j]
                    t = sbuf[j, pl.ds(80, 16)] + dbuf[j, pl.ds(96, 16)]
                    eej = jnp.where(hm, jnp.exp(_lrelu(t) - cv), 0.0)
                    plsc.addupdate(den.at[pl.ds(aj * 16, 16)], eej)
                    xv = [sbuf[j, pl.ds(vv * 16, 16)] for vv in range(5)]
                    for h in range(_H1):
                        ehv = jnp.full((16,), eej[h], jnp.float32)
                        for vv in range(5):
                            plsc.addupdate(
                                acc.at[pl.ds(aj * 800 + h * 80 + vv * 16,
                                             16)],
                                ehv * xv[vv])
                return 0

            lax.fori_loop(0, ngroups, group_body, 0)
            return 0

        lax.fori_loop(0, _E // _CH, chunk_body, 0)

        pltpu.sync_copy(acc.at[pl.ds(0, 128 * 800)],
                        z1.at[pl.ds(base * 800, 128 * 800)])
        pltpu.sync_copy(den.at[pl.ds(0, 128 * 16)],
                        den1.at[pl.ds(base * 16, 128 * 16)])
        return 0

    lax.fori_loop(0, 4, pass_body, 0)


def _sc_layer1(x_packed, pmax, src, dst):
    mesh = plsc.VectorSubcoreMesh(core_axis_name="c", subcore_axis_name="s")
    fn = functools.partial(
        pl.kernel, mesh=mesh,
        compiler_params=pltpu.CompilerParams(needs_layout_passes=False),
        out_type=[
            jax.ShapeDtypeStruct((_N * 800,), jnp.float32),
            jax.ShapeDtypeStruct((_N * 16,), jnp.float32),
        ],
        scratch_types=[
            pltpu.VMEM((129 * 800,), jnp.float32),
            pltpu.VMEM((129 * 16,), jnp.float32),
            pltpu.VMEM((_CH,), jnp.int32),
            pltpu.VMEM((_CH,), jnp.int32),
            pltpu.VMEM((_CH + 16,), jnp.int32),
            pltpu.VMEM((_CH + 16,), jnp.int32),
            pltpu.VMEM((16, 128), jnp.float32),
            pltpu.VMEM((16, 128), jnp.float32),
            pltpu.VMEM((512,), jnp.float32),
            pltpu.SemaphoreType.DMA,
            pltpu.SemaphoreType.DMA,
        ],
    )(_sc_l1_body)
    return fn(x_packed, pmax.reshape(32 * 16), src, dst)


def _agg1_body(z1_ref, den1_ref, b1_ref, W1h_ref, W2_ref, al2_ref,
               ar2_ref, feat2_ref, el2_ref, er2_ref, pmax_ref):
    blk = z1_ref.shape[0]
    den1 = den1_ref[...]
    feat2 = jnp.zeros((blk, _F2), jnp.float32)
    for h in range(_H1):
        zh = z1_ref[:, pl.ds(h * 80, 74)]
        oh = jax.lax.dot_general(
            zh, W1h_ref[h], (((1,), (0,)), ((), ())),
            preferred_element_type=jnp.float32)
        oh = oh / (den1[:, h:h + 1] + 1e-9)
        oh = oh + b1_ref[0, pl.ds(h * _F1, _F1)].reshape(1, _F1)
        h1h = jnp.where(oh > 0, oh, jnp.exp(jnp.minimum(oh, 0.0)) - 1.0)
        feat2 = feat2 + jax.lax.dot_general(
            h1h, W2_ref[pl.ds(h * _F1, _F1), :],
            (((1,), (0,)), ((), ())), preferred_element_type=jnp.float32)
    feat2_ref[...] = feat2
    el2 = jax.lax.dot_general(
        feat2, al2_ref[...], (((1,), (1,)), ((), ())),
        preferred_element_type=jnp.float32)
    er2 = jax.lax.dot_general(
        feat2, ar2_ref[...], (((1,), (1,)), ((), ())),
        preferred_element_type=jnp.float32)
    el2_ref[...] = el2
    er2_ref[...] = er2
    lane = lax.broadcasted_iota(jnp.int32, (1, 1, 16), 2)
    mel = jnp.max(el2)
    mer = jnp.max(er2)
    pmax_ref[...] = jnp.where(lane == 0, mel,
                              jnp.where(lane == 1, mer, -1e30))


def _agg1(z1, den1, b1, W1h, W2, al2, ar2):
    nblk = 32
    blk = _N // nblk
    return pl.pallas_call(
        _agg1_body,
        grid=(nblk,),
        in_specs=[
            pl.BlockSpec((blk, 800), lambda g: (g, 0)),
            pl.BlockSpec((blk, 16), lambda g: (g, 0)),
            pl.BlockSpec((1, _H1 * _F1), lambda g: (0, 0)),
            pl.BlockSpec((_H1, 74, 74), lambda g: (0, 0, 0)),
            pl.BlockSpec((_H1 * _F1, _F2), lambda g: (0, 0)),
            pl.BlockSpec((1, _F2), lambda g: (0, 0)),
            pl.BlockSpec((1, _F2), lambda g: (0, 0)),
        ],
        out_specs=[
            pl.BlockSpec((blk, _F2), lambda g: (g, 0)),
            pl.BlockSpec((blk, 1), lambda g: (g, 0)),
            pl.BlockSpec((blk, 1), lambda g: (g, 0)),
            pl.BlockSpec((1, 1, 16), lambda g: (g, 0, 0)),
        ],
        out_shape=[
            jax.ShapeDtypeStruct((_N, _F2), jnp.float32),
            jax.ShapeDtypeStruct((_N, 1), jnp.float32),
            jax.ShapeDtypeStruct((_N, 1), jnp.float32),
            jax.ShapeDtypeStruct((nblk, 1, 16), jnp.float32),
        ],
    )(z1.reshape(_N, 800), den1.reshape(_N, 16), b1.reshape(1, -1),
      W1h, W2, al2, ar2)


def kernel(node_feats, edge_index, node_graph_ids, target, dist, W1, al1,
           ar1, b1, W2, al2, ar2, b2, atomW, atomb, protW, protb, distW,
           distb, d1W, d1b, d2W, d2b, outW, outb):
    src, dst = edge_index[0], edge_index[1]
    W1r = W1.reshape(74, _H1, _F1)
    Wel1 = jnp.einsum('fhg,hg->fh', W1r, al1)
    Wer1 = jnp.einsum('fhg,hg->fh', W1r, ar1)
    W1h = W1r.transpose(1, 0, 2)
    x_packed, pmax1 = _attn1(node_feats, Wel1, Wer1)
    z1, den1 = _sc_layer1(x_packed, pmax1, src, dst)
    feat2, el2, er2, pmax = _agg1(z1, den1, b1, W1h, W2, al2, ar2)
    z2, den2 = _sc_layer2(feat2, el2, er2, pmax, src, dst)
    return _head(z2, den2, b2, target, dist, atomW, atomb, protW, protb,
                 distW, distb, d1W, d1b, d2W, d2b, outW, outb)
